# Initial kernel scaffold; baseline (speedup 1.0000x reference)
import functools

import jax
import jax.numpy as jnp
from jax import lax
from jax.experimental import pallas as pl
from jax.experimental.pallas import tpu as pltpu
from jax.experimental.pallas import tpu_sc as plsc

CHUNK = 128


def _make_gather(nw, nc, ch, k, d, n):
    g_rows = k * CHUNK
    rows_per_w = ch * CHUNK
    groups = ch // k
    mesh = plsc.VectorSubcoreMesh(core_axis_name="c", subcore_axis_name="s")

    @functools.partial(
        pl.kernel,
        out_type=jax.ShapeDtypeStruct((n, d), jnp.float32),
        mesh=mesh,
        scratch_types=[
            pltpu.VMEM((ch, CHUNK), jnp.int32),
            pltpu.VMEM((2, g_rows, d), jnp.float32),
            pltpu.SemaphoreType.DMA,
        ],
    )
    def gather_kernel(idx_hbm, table_hbm, out_hbm, idx_v, rows_v, sem):
        wid = lax.axis_index("s") * nc + lax.axis_index("c")
        base = wid * rows_per_w
        pltpu.sync_copy(idx_hbm.at[wid], idx_v)

        def fire_group(g, slot):
            for j in range(k):
                pltpu.async_copy(
                    table_hbm.at[idx_v.at[g * k + j]],
                    rows_v.at[slot, pl.ds(j * CHUNK, CHUNK)],
                    sem,
                )

        fire_group(0, 0)

        def body(g, carry):
            slot = lax.rem(g, 2)

            @pl.when(g + 1 < groups)
            def _():
                fire_group(g + 1, 1 - slot)

            pltpu.make_async_copy(
                out_hbm.at[pl.ds(0, g_rows)], rows_v.at[slot], sem
            ).wait()
            pltpu.sync_copy(
                rows_v.at[slot], out_hbm.at[pl.ds(base + g * g_rows, g_rows)]
            )
            return carry

        lax.fori_loop(0, groups, body, 0)

    return gather_kernel


def kernel(indices, weight):
    b, f = indices.shape
    v, d = weight.shape
    n = b * f
    info = plsc.get_sparse_core_info()
    nc, ns = info.num_cores, info.num_subcores
    nw = nc * ns
    assert n % (nw * CHUNK) == 0
    ch = n // (nw * CHUNK)
    k = 4
    while ch % k:
        k -= 1
    idx3 = indices.reshape(nw, ch, CHUNK)
    out = _make_gather(nw, nc, ch, k, d, n)(idx3, weight)
    return out.reshape(b, f, d)


# speedup vs baseline: 4.5465x; 4.5465x over previous
import functools

import jax
import jax.numpy as jnp
from jax import lax
from jax.experimental import pallas as pl
from jax.experimental.pallas import tpu as pltpu
from jax.experimental.pallas import tpu_sc as plsc

CHUNK = 128


def _make_gather(nw, nc, ch, k, d, n):
    g_rows = k * CHUNK
    rows_per_w = ch * CHUNK
    groups = ch // k
    mesh = plsc.VectorSubcoreMesh(core_axis_name="c", subcore_axis_name="s")

    @functools.partial(
        pl.kernel,
        out_type=jax.ShapeDtypeStruct((n, d), jnp.float32),
        mesh=mesh,
        scratch_types=[
            pltpu.VMEM((ch, CHUNK), jnp.int32),
            pltpu.VMEM((2, g_rows, d), jnp.float32),
            pltpu.SemaphoreType.DMA,
        ],
        compiler_params=pltpu.CompilerParams(use_tc_tiling_on_sc=False),
    )
    def gather_kernel(idx_hbm, table_hbm, out_hbm, idx_v, rows_v, sem):
        wid = lax.axis_index("s") * nc + lax.axis_index("c")
        base = wid * rows_per_w
        pltpu.sync_copy(idx_hbm.at[wid], idx_v)

        def fire_group(g, slot):
            for j in range(k):
                pltpu.async_copy(
                    table_hbm.at[idx_v.at[g * k + j]],
                    rows_v.at[slot, pl.ds(j * CHUNK, CHUNK)],
                    sem,
                )

        fire_group(0, 0)

        def body(g, carry):
            slot = lax.rem(g, 2)

            @pl.when(g + 1 < groups)
            def _():
                fire_group(g + 1, 1 - slot)

            pltpu.make_async_copy(
                out_hbm.at[pl.ds(0, g_rows)], rows_v.at[slot], sem
            ).wait()
            pltpu.sync_copy(
                rows_v.at[slot], out_hbm.at[pl.ds(base + g * g_rows, g_rows)]
            )
            return carry

        lax.fori_loop(0, groups, body, 0)

    return gather_kernel


def kernel(indices, weight):
    b, f = indices.shape
    v, d = weight.shape
    n = b * f
    info = plsc.get_sparse_core_info()
    nc, ns = info.num_cores, info.num_subcores
    nw = nc * ns
    assert n % (nw * CHUNK) == 0
    ch = n // (nw * CHUNK)
    k = 4
    while ch % k:
        k -= 1
    idx3 = indices.reshape(nw, ch, CHUNK)
    out = _make_gather(nw, nc, ch, k, d, n)(idx3, weight)
    return out.reshape(b, f, d)


# speedup vs baseline: 4.5528x; 1.0014x over previous
import functools

import jax
import jax.numpy as jnp
from jax import lax
from jax.experimental import pallas as pl
from jax.experimental.pallas import tpu as pltpu
from jax.experimental.pallas import tpu_sc as plsc

CHUNK = 128


def _make_gather(nw, nc, ch, k, d, n):
    g_rows = k * CHUNK
    rows_per_w = ch * CHUNK
    groups = ch // k
    mesh = plsc.VectorSubcoreMesh(core_axis_name="c", subcore_axis_name="s")

    @functools.partial(
        pl.kernel,
        out_type=jax.ShapeDtypeStruct((n, d), jnp.float32),
        mesh=mesh,
        scratch_types=[
            pltpu.VMEM((ch, CHUNK), jnp.int32),
            pltpu.VMEM((2, g_rows, d), jnp.float32),
            pltpu.SemaphoreType.DMA,
        ],
        compiler_params=pltpu.CompilerParams(use_tc_tiling_on_sc=False),
    )
    def gather_kernel(idx_hbm, table_hbm, out_hbm, idx_v, rows_v, sem):
        wid = lax.axis_index("s") * nc + lax.axis_index("c")
        base = wid * rows_per_w
        pltpu.sync_copy(idx_hbm.at[wid], idx_v)

        def fire_group(g, slot):
            for j in range(k):
                pltpu.async_copy(
                    table_hbm.at[idx_v.at[g * k + j]],
                    rows_v.at[slot, pl.ds(j * CHUNK, CHUNK)],
                    sem,
                )

        fire_group(0, 0)

        def body(g, carry):
            slot = lax.rem(g, 2)

            @pl.when(g + 1 < groups)
            def _():
                fire_group(g + 1, 1 - slot)

            pltpu.make_async_copy(
                out_hbm.at[pl.ds(0, g_rows)], rows_v.at[slot], sem
            ).wait()
            pltpu.sync_copy(
                rows_v.at[slot], out_hbm.at[pl.ds(base + g * g_rows, g_rows)]
            )
            return carry

        lax.fori_loop(0, groups, body, 0)

    return gather_kernel


def kernel(indices, weight):
    b, f = indices.shape
    v, d = weight.shape
    n = b * f
    info = plsc.get_sparse_core_info()
    nc, ns = info.num_cores, info.num_subcores
    nw = nc * ns
    assert n % (nw * CHUNK) == 0
    ch = n // (nw * CHUNK)
    k = 10
    while ch % k:
        k -= 1
    idx3 = indices.reshape(nw, ch, CHUNK)
    out = _make_gather(nw, nc, ch, k, d, n)(idx3, weight)
    return out.reshape(b, f, d)
